# baseline (speedup 1.0000x reference)
import jax
import jax.numpy as jnp
from jax import lax
from jax.experimental import pallas as pl
from jax.experimental.pallas import tpu as pltpu
from jax.experimental.pallas import tpu_sc as plsc

N_NODES = 10000
N_EDGES = 320000
D = 128

NC = 2
NS = 16
NW = NC * NS
EPW = N_EDGES // NW
CHUNK = 80
NCHUNK = EPW // CHUNK
IBLK = 25
NIB = NCHUNK // IBLK
NACC = 10000
ACC_T = 10
ACC_R = NACC // ACC_T
NPAD_H = 10240
SEG = NPAD_H // NS


def _sc_body(x_hbm, ei_hbm, acc_out, deg_out,
             src_v, dst_v, rows_a, rows_b, rows_c, hist_v, red_v, tmp_v,
             acc_sh, hists_sh, sem_a, sem_b, sem_c,
             sem_sa, sem_sb, sem_sc):
    cid = lax.axis_index("c")
    sid = lax.axis_index("s")
    wid = sid * NC + cid

    z16 = jnp.zeros((16,), jnp.float32)

    def zhist(i, _):
        for c in range(4):
            hist_v[pl.ds(i * 64 + c * 16, 16)] = z16
        return 0

    lax.fori_loop(0, NPAD_H // 64, zhist, 0)
    for c in range(SEG // 16):
        red_v[pl.ds(c * 16, 16)] = z16

    def zrow(i, _):
        for c in range(D // 16):
            rows_a[i, pl.ds(c * 16, 16)] = z16
        return 0

    lax.fori_loop(0, CHUNK, zrow, 0)

    @pl.when(sid < ACC_T)
    def _():
        for k in range(ACC_R // CHUNK):
            pltpu.sync_copy(
                rows_a, acc_sh.at[pl.ds(sid * ACC_R + k * CHUNK, CHUNK)])
        pltpu.sync_copy(rows_a.at[pl.ds(0, ACC_R % CHUNK)],
                        acc_sh.at[pl.ds(sid * ACC_R + ACC_R - ACC_R % CHUNK,
                                        ACC_R % CHUNK)])

    plsc.subcore_barrier()

    ones16 = jnp.full((16,), 1.0, jnp.float32)
    bufs = (rows_a, rows_b, rows_c)
    gsems = (sem_a, sem_b, sem_c)
    ssems = (sem_sa, sem_sb, sem_sc)

    def block(ib, _):
        pltpu.sync_copy(ei_hbm.at[0, wid, ib], src_v)
        pltpu.sync_copy(ei_hbm.at[1, wid, ib], dst_v)
        pltpu.async_copy(x_hbm.at[src_v.at[0]], rows_a, sem_a)

        def step(j, _):
            @pl.when(j >= 1)
            def _():
                for b in range(3):
                    @pl.when(lax.rem(j - 1, 3) == b)
                    def _(b=b):
                        pltpu.make_async_copy(
                            bufs[b], acc_sh.at[dst_v.at[j]],
                            ssems[b]).wait()

            @pl.when(j + 1 < IBLK)
            def _():
                for b in range(3):
                    @pl.when(lax.rem(j + 1, 3) == b)
                    def _(b=b):
                        pltpu.async_copy(
                            x_hbm.at[src_v.at[j + 1]], bufs[b], gsems[b])

            for k in range(CHUNK // 16):
                idx = dst_v[j, pl.ds(k * 16, 16)]
                plsc.addupdate_scatter(hist_v, [idx], ones16)

            for b in range(3):
                @pl.when(lax.rem(j, 3) == b)
                def _(b=b):
                    pltpu.make_async_copy(
                        x_hbm.at[src_v.at[j]], bufs[b], gsems[b]).wait()
                    pltpu.async_copy(bufs[b], acc_sh.at[dst_v.at[j]],
                                     ssems[b], add=True)

            return 0

        lax.fori_loop(0, IBLK, step, 0)
        pltpu.make_async_copy(rows_a, acc_sh.at[dst_v.at[0]], ssems[0]).wait()
        return 0

    lax.fori_loop(0, NIB, block, 0)

    def phase(p, _):
        pub = lax.rem(sid + p, NS)
        pltpu.sync_copy(hist_v.at[pl.ds(pub * SEG, SEG)], hists_sh.at[sid])
        plsc.subcore_barrier()
        slot = lax.rem(sid - p + NS, NS)
        pltpu.sync_copy(hists_sh.at[slot], tmp_v)
        for c in range(SEG // 16):
            sl = pl.ds(c * 16, 16)
            red_v[sl] = red_v[sl] + tmp_v[sl]
        plsc.subcore_barrier()
        return 0

    lax.fori_loop(0, NS, phase, 0)

    pltpu.sync_copy(red_v, deg_out.at[cid, pl.ds(sid * SEG, SEG)])

    @pl.when(sid < ACC_T)
    def _():
        pltpu.sync_copy(acc_sh.at[pl.ds(sid * ACC_R, ACC_R)],
                        acc_out.at[cid, pl.ds(sid * ACC_R, ACC_R)])


def _tc_body(p_ref, d_ref, x_ref, w_ref, b_ref, o_ref):
    p = p_ref[0] + p_ref[1]
    dg = d_ref[0] + d_ref[1]
    agg = p / jnp.maximum(dg, 1.0)
    h = jnp.dot(agg, w_ref[...], preferred_element_type=jnp.float32) + b_ref[...]
    o_ref[...] = jnp.maximum(h, 0.0) + x_ref[...]


def kernel(x, edge_index, W, b):
    ei = edge_index.astype(jnp.int32).reshape(2, NW, NIB, IBLK, CHUNK)

    mesh = plsc.VectorSubcoreMesh(core_axis_name="c", subcore_axis_name="s")
    acc_p, deg_p = pl.kernel(
        _sc_body,
        out_type=(
            jax.ShapeDtypeStruct((NC, NACC, D), jnp.float32),
            jax.ShapeDtypeStruct((NC, NPAD_H), jnp.float32),
        ),
        mesh=mesh,
        compiler_params=pltpu.CompilerParams(needs_layout_passes=False),
        scratch_types=[
            pltpu.VMEM((IBLK, CHUNK), jnp.int32),
            pltpu.VMEM((IBLK, CHUNK), jnp.int32),
            pltpu.VMEM((CHUNK, D), jnp.float32),
            pltpu.VMEM((CHUNK, D), jnp.float32),
            pltpu.VMEM((CHUNK, D), jnp.float32),
            pltpu.VMEM((NPAD_H,), jnp.float32),
            pltpu.VMEM((SEG,), jnp.float32),
            pltpu.VMEM((SEG,), jnp.float32),
            pltpu.VMEM_SHARED((NACC, D), jnp.float32),
            pltpu.VMEM_SHARED((NS, SEG), jnp.float32),
            pltpu.SemaphoreType.DMA,
            pltpu.SemaphoreType.DMA,
            pltpu.SemaphoreType.DMA,
            pltpu.SemaphoreType.DMA,
            pltpu.SemaphoreType.DMA,
            pltpu.SemaphoreType.DMA,
        ],
    )(x, ei)

    deg_flat = deg_p.reshape(NC, NPAD_H, 1)

    R = 1000
    grid = (N_NODES // R,)
    h = pl.pallas_call(
        _tc_body,
        grid=grid,
        in_specs=[
            pl.BlockSpec((NC, R, D), lambda i: (0, i, 0)),
            pl.BlockSpec((NC, R, 1), lambda i: (0, i, 0)),
            pl.BlockSpec((R, D), lambda i: (i, 0)),
            pl.BlockSpec((D, D), lambda i: (0, 0)),
            pl.BlockSpec((1, D), lambda i: (0, 0)),
        ],
        out_specs=pl.BlockSpec((R, D), lambda i: (i, 0)),
        out_shape=jax.ShapeDtypeStruct((N_NODES, D), jnp.float32),
    )(acc_p, deg_flat, x, W, b.reshape(1, D))
    return h


# speedup vs baseline: 1.1112x; 1.1112x over previous
import jax
import jax.numpy as jnp
from jax import lax
from jax.experimental import pallas as pl
from jax.experimental.pallas import tpu as pltpu
from jax.experimental.pallas import tpu_sc as plsc

N_NODES = 10000
N_EDGES = 320000
D = 128

NC = 2
NS = 16
NW = NC * NS
EPW = N_EDGES // NW
CHUNK = 80
NCHUNK = EPW // CHUNK
IBLK = 25
NIB = NCHUNK // IBLK
NACC = 10000
ACC_T = 10
ACC_R = NACC // ACC_T
NPAD_H = 10240
SEG = NPAD_H // NS


def _sc_body(x_hbm, ei_hbm, acc_out, deg_out,
             src_v, dst_v, rows_a, rows_b, rows_c, hist_v, red_v, tmp_v,
             acc_sh, hists_sh, sem_a, sem_b, sem_c,
             sem_sa, sem_sb, sem_sc):
    cid = lax.axis_index("c")
    sid = lax.axis_index("s")
    wid = sid * NC + cid

    z16 = jnp.zeros((16,), jnp.float32)

    def zhist(i, _):
        for c in range(4):
            hist_v[pl.ds(i * 64 + c * 16, 16)] = z16
        return 0

    lax.fori_loop(0, NPAD_H // 64, zhist, 0)
    for c in range(SEG // 16):
        red_v[pl.ds(c * 16, 16)] = z16

    def zrow(i, _):
        for c in range(D // 16):
            rows_a[i, pl.ds(c * 16, 16)] = z16
        return 0

    lax.fori_loop(0, CHUNK, zrow, 0)

    @pl.when(sid < ACC_T)
    def _():
        for k in range(ACC_R // CHUNK):
            pltpu.sync_copy(
                rows_a, acc_sh.at[pl.ds(sid * ACC_R + k * CHUNK, CHUNK)])
        pltpu.sync_copy(rows_a.at[pl.ds(0, ACC_R % CHUNK)],
                        acc_sh.at[pl.ds(sid * ACC_R + ACC_R - ACC_R % CHUNK,
                                        ACC_R % CHUNK)])

    plsc.subcore_barrier()

    ones16 = jnp.full((16,), 1.0, jnp.float32)
    bufs = (rows_a, rows_b, rows_c)
    gsems = (sem_a, sem_b, sem_c)
    ssems = (sem_sa, sem_sb, sem_sc)

    def block(ib, _):
        pltpu.sync_copy(ei_hbm.at[0, wid, ib], src_v)
        pltpu.sync_copy(ei_hbm.at[1, wid, ib], dst_v)
        pltpu.async_copy(x_hbm.at[src_v.at[0]], rows_a, sem_a)

        def step(j, _):
            @pl.when(j + 1 < IBLK)
            def _():
                for b in range(3):
                    @pl.when(lax.rem(j + 1, 3) == b)
                    def _(b=b):
                        @pl.when(j >= 2)
                        def _(b=b):
                            pltpu.make_async_copy(
                                bufs[b], acc_sh.at[dst_v.at[j]],
                                ssems[b]).wait()
                        pltpu.async_copy(
                            x_hbm.at[src_v.at[j + 1]], bufs[b], gsems[b])

            for k in range(CHUNK // 16):
                idx = dst_v[j, pl.ds(k * 16, 16)]
                plsc.addupdate_scatter(hist_v, [idx], ones16)

            for b in range(3):
                @pl.when(lax.rem(j, 3) == b)
                def _(b=b):
                    pltpu.make_async_copy(
                        x_hbm.at[src_v.at[j]], bufs[b], gsems[b]).wait()
                    pltpu.async_copy(bufs[b], acc_sh.at[dst_v.at[j]],
                                     ssems[b], add=True)

            return 0

        lax.fori_loop(0, IBLK, step, 0)
        for b in (1, 2, 0):
            pltpu.make_async_copy(
                bufs[b], acc_sh.at[dst_v.at[0]], ssems[b]).wait()
        return 0

    lax.fori_loop(0, NIB, block, 0)

    def phase(p, _):
        pub = lax.rem(sid + p, NS)
        pltpu.sync_copy(hist_v.at[pl.ds(pub * SEG, SEG)], hists_sh.at[sid])
        plsc.subcore_barrier()
        slot = lax.rem(sid - p + NS, NS)
        pltpu.sync_copy(hists_sh.at[slot], tmp_v)
        for c in range(SEG // 16):
            sl = pl.ds(c * 16, 16)
            red_v[sl] = red_v[sl] + tmp_v[sl]
        plsc.subcore_barrier()
        return 0

    lax.fori_loop(0, NS, phase, 0)

    pltpu.sync_copy(red_v, deg_out.at[cid, pl.ds(sid * SEG, SEG)])

    @pl.when(sid < ACC_T)
    def _():
        pltpu.sync_copy(acc_sh.at[pl.ds(sid * ACC_R, ACC_R)],
                        acc_out.at[cid, pl.ds(sid * ACC_R, ACC_R)])


def _tc_body(p_ref, d_ref, x_ref, w_ref, b_ref, o_ref):
    p = p_ref[0] + p_ref[1]
    dg = d_ref[0] + d_ref[1]
    agg = p / jnp.maximum(dg, 1.0)
    h = jnp.dot(agg, w_ref[...], preferred_element_type=jnp.float32) + b_ref[...]
    o_ref[...] = jnp.maximum(h, 0.0) + x_ref[...]


def kernel(x, edge_index, W, b):
    ei = edge_index.astype(jnp.int32).reshape(2, NW, NIB, IBLK, CHUNK)

    mesh = plsc.VectorSubcoreMesh(core_axis_name="c", subcore_axis_name="s")
    acc_p, deg_p = pl.kernel(
        _sc_body,
        out_type=(
            jax.ShapeDtypeStruct((NC, NACC, D), jnp.float32),
            jax.ShapeDtypeStruct((NC, NPAD_H), jnp.float32),
        ),
        mesh=mesh,
        compiler_params=pltpu.CompilerParams(needs_layout_passes=False),
        scratch_types=[
            pltpu.VMEM((IBLK, CHUNK), jnp.int32),
            pltpu.VMEM((IBLK, CHUNK), jnp.int32),
            pltpu.VMEM((CHUNK, D), jnp.float32),
            pltpu.VMEM((CHUNK, D), jnp.float32),
            pltpu.VMEM((CHUNK, D), jnp.float32),
            pltpu.VMEM((NPAD_H,), jnp.float32),
            pltpu.VMEM((SEG,), jnp.float32),
            pltpu.VMEM((SEG,), jnp.float32),
            pltpu.VMEM_SHARED((NACC, D), jnp.float32),
            pltpu.VMEM_SHARED((NS, SEG), jnp.float32),
            pltpu.SemaphoreType.DMA,
            pltpu.SemaphoreType.DMA,
            pltpu.SemaphoreType.DMA,
            pltpu.SemaphoreType.DMA,
            pltpu.SemaphoreType.DMA,
            pltpu.SemaphoreType.DMA,
        ],
    )(x, ei)

    deg_flat = deg_p.reshape(NC, NPAD_H, 1)

    R = 1000
    grid = (N_NODES // R,)
    h = pl.pallas_call(
        _tc_body,
        grid=grid,
        in_specs=[
            pl.BlockSpec((NC, R, D), lambda i: (0, i, 0)),
            pl.BlockSpec((NC, R, 1), lambda i: (0, i, 0)),
            pl.BlockSpec((R, D), lambda i: (i, 0)),
            pl.BlockSpec((D, D), lambda i: (0, 0)),
            pl.BlockSpec((1, D), lambda i: (0, 0)),
        ],
        out_specs=pl.BlockSpec((R, D), lambda i: (i, 0)),
        out_shape=jax.ShapeDtypeStruct((N_NODES, D), jnp.float32),
    )(acc_p, deg_flat, x, W, b.reshape(1, D))
    return h


# speedup vs baseline: 1.1679x; 1.0510x over previous
import jax
import jax.numpy as jnp
from jax import lax
from jax.experimental import pallas as pl
from jax.experimental.pallas import tpu as pltpu
from jax.experimental.pallas import tpu_sc as plsc

N_NODES = 10000
N_EDGES = 320000
D = 128

NC = 2
NS = 16
NW = NC * NS
EPW = N_EDGES // NW
CHUNK = 80
NCHUNK = EPW // CHUNK
IBLK = 25
NIB = NCHUNK // IBLK
NACC = 10000
ACC_T = 10
ACC_R = NACC // ACC_T
NPAD_H = 10240
SEG = NPAD_H // NS


def _sc_body(x_hbm, ei_hbm, acc_out, deg_out,
             src_v, dst_v, rows_a, rows_b, rows_c, hist_v, red_v, tmp_v,
             acc_sh, hists_sh, sem_a, sem_b, sem_c,
             sem_sa, sem_sb, sem_sc):
    cid = lax.axis_index("c")
    sid = lax.axis_index("s")
    wid = sid * NC + cid

    z16 = jnp.zeros((16,), jnp.float32)

    def zhist(i, _):
        for c in range(4):
            hist_v[pl.ds(i * 64 + c * 16, 16)] = z16
        return 0

    lax.fori_loop(0, NPAD_H // 64, zhist, 0)
    for c in range(SEG // 16):
        red_v[pl.ds(c * 16, 16)] = z16

    def zrow(i, _):
        for c in range(D // 16):
            rows_a[i, pl.ds(c * 16, 16)] = z16
        return 0

    lax.fori_loop(0, CHUNK, zrow, 0)

    @pl.when(sid < ACC_T)
    def _():
        for k in range(ACC_R // CHUNK):
            pltpu.sync_copy(
                rows_a, acc_sh.at[pl.ds(sid * ACC_R + k * CHUNK, CHUNK)])
        pltpu.sync_copy(rows_a.at[pl.ds(0, ACC_R % CHUNK)],
                        acc_sh.at[pl.ds(sid * ACC_R + ACC_R - ACC_R % CHUNK,
                                        ACC_R % CHUNK)])

    plsc.subcore_barrier()

    ones16 = jnp.full((16,), 1.0, jnp.float32)
    bufs = (rows_a, rows_b, rows_c)
    gsems = (sem_a, sem_b, sem_c)
    ssems = (sem_sa, sem_sb, sem_sc)

    def block(ib, _):
        pltpu.sync_copy(ei_hbm.at[0, wid, ib], src_v)
        pltpu.sync_copy(ei_hbm.at[1, wid, ib], dst_v)
        pltpu.async_copy(x_hbm.at[src_v.at[0]], rows_a, sem_a)

        def step(j, _):
            @pl.when(j + 1 < IBLK)
            def _():
                for b in range(3):
                    @pl.when(lax.rem(j + 1, 3) == b)
                    def _(b=b):
                        @pl.when(j >= 2)
                        def _(b=b):
                            pltpu.make_async_copy(
                                bufs[b], acc_sh.at[dst_v.at[j]],
                                ssems[b]).wait()
                        pltpu.async_copy(
                            x_hbm.at[src_v.at[j + 1]], bufs[b], gsems[b])

            for k in range(CHUNK // 16):
                idx = dst_v[j, pl.ds(k * 16, 16)]
                plsc.addupdate_scatter(hist_v, [idx], ones16)

            for b in range(3):
                @pl.when(lax.rem(j, 3) == b)
                def _(b=b):
                    pltpu.make_async_copy(
                        x_hbm.at[src_v.at[j]], bufs[b], gsems[b]).wait()
                    pltpu.async_copy(bufs[b], acc_sh.at[dst_v.at[j]],
                                     ssems[b], add=True)

            return 0

        lax.fori_loop(0, IBLK, step, 0)
        for b in (1, 2, 0):
            pltpu.make_async_copy(
                bufs[b], acc_sh.at[dst_v.at[0]], ssems[b]).wait()
        return 0

    lax.fori_loop(0, NIB, block, 0)

    def phase(p, _):
        pub = lax.rem(sid + p, NS)
        pltpu.sync_copy(hist_v.at[pl.ds(pub * SEG, SEG)], hists_sh.at[sid])
        plsc.subcore_barrier()
        slot = lax.rem(sid - p + NS, NS)
        pltpu.sync_copy(hists_sh.at[slot], tmp_v)
        for c in range(SEG // 16):
            sl = pl.ds(c * 16, 16)
            red_v[sl] = red_v[sl] + tmp_v[sl]
        plsc.subcore_barrier()
        return 0

    lax.fori_loop(0, NS, phase, 0)

    pltpu.sync_copy(red_v, deg_out.at[cid, pl.ds(sid * SEG, SEG)])

    @pl.when(sid < ACC_T)
    def _():
        pltpu.sync_copy(acc_sh.at[pl.ds(sid * ACC_R, ACC_R)],
                        acc_out.at[cid, pl.ds(sid * ACC_R, ACC_R)])


def _tc_body(p_ref, d_ref, x_ref, w_ref, b_ref, o_ref):
    p = p_ref[0] + p_ref[1]
    dgr = (d_ref[0] + d_ref[1]).reshape(1, -1)
    dg = jnp.transpose(dgr, (1, 0))
    agg = p / jnp.maximum(dg, 1.0)
    h = jnp.dot(agg, w_ref[...], preferred_element_type=jnp.float32) + b_ref[...]
    o_ref[...] = jnp.maximum(h, 0.0) + x_ref[...]


def kernel(x, edge_index, W, b):
    ei = edge_index.astype(jnp.int32).reshape(2, NW, NIB, IBLK, CHUNK)

    mesh = plsc.VectorSubcoreMesh(core_axis_name="c", subcore_axis_name="s")
    acc_p, deg_p = pl.kernel(
        _sc_body,
        out_type=(
            jax.ShapeDtypeStruct((NC, NACC, D), jnp.float32),
            jax.ShapeDtypeStruct((NC, NPAD_H), jnp.float32),
        ),
        mesh=mesh,
        compiler_params=pltpu.CompilerParams(needs_layout_passes=False),
        scratch_types=[
            pltpu.VMEM((IBLK, CHUNK), jnp.int32),
            pltpu.VMEM((IBLK, CHUNK), jnp.int32),
            pltpu.VMEM((CHUNK, D), jnp.float32),
            pltpu.VMEM((CHUNK, D), jnp.float32),
            pltpu.VMEM((CHUNK, D), jnp.float32),
            pltpu.VMEM((NPAD_H,), jnp.float32),
            pltpu.VMEM((SEG,), jnp.float32),
            pltpu.VMEM((SEG,), jnp.float32),
            pltpu.VMEM_SHARED((NACC, D), jnp.float32),
            pltpu.VMEM_SHARED((NS, SEG), jnp.float32),
            pltpu.SemaphoreType.DMA,
            pltpu.SemaphoreType.DMA,
            pltpu.SemaphoreType.DMA,
            pltpu.SemaphoreType.DMA,
            pltpu.SemaphoreType.DMA,
            pltpu.SemaphoreType.DMA,
        ],
    )(x, ei)

    R = 1024
    grid = ((N_NODES + R - 1) // R,)
    h = pl.pallas_call(
        _tc_body,
        grid=grid,
        in_specs=[
            pl.BlockSpec((NC, R, D), lambda i: (0, i, 0)),
            pl.BlockSpec((NC, R), lambda i: (0, i)),
            pl.BlockSpec((R, D), lambda i: (i, 0)),
            pl.BlockSpec((D, D), lambda i: (0, 0)),
            pl.BlockSpec((1, D), lambda i: (0, 0)),
        ],
        out_specs=pl.BlockSpec((R, D), lambda i: (i, 0)),
        out_shape=jax.ShapeDtypeStruct((N_NODES, D), jnp.float32),
    )(acc_p, deg_p, x, W, b.reshape(1, D))
    return h


# speedup vs baseline: 1.1785x; 1.0090x over previous
import jax
import jax.numpy as jnp
from jax import lax
from jax.experimental import pallas as pl
from jax.experimental.pallas import tpu as pltpu
from jax.experimental.pallas import tpu_sc as plsc

N_NODES = 10000
N_EDGES = 320000
D = 128

NC = 2
NS = 16
NW = NC * NS
EPW = N_EDGES // NW
CHUNK = 80
NCHUNK = EPW // CHUNK
IBLK = 25
NIB = NCHUNK // IBLK
NACC = 10000
ACC_T = 10
ACC_R = NACC // ACC_T
NPAD_H = 10240
SEG = NPAD_H // NS


def _sc_body(x_hbm, ei_hbm, acc_out, deg_out,
             src_v, dst_v, rows_a, rows_b, rows_c, hist_v, red_v, tmp_v,
             acc_sh, hists_sh, sem_a, sem_b, sem_c,
             sem_sa, sem_sb, sem_sc):
    cid = lax.axis_index("c")
    sid = lax.axis_index("s")
    wid = sid * NC + cid

    z16 = jnp.zeros((16,), jnp.float32)

    def zhist(i, _):
        for c in range(4):
            hist_v[pl.ds(i * 64 + c * 16, 16)] = z16
        return 0

    lax.fori_loop(0, NPAD_H // 64, zhist, 0)
    for c in range(SEG // 16):
        red_v[pl.ds(c * 16, 16)] = z16

    def zrow(i, _):
        for c in range(D // 16):
            rows_a[i, pl.ds(c * 16, 16)] = z16
        return 0

    lax.fori_loop(0, CHUNK, zrow, 0)

    @pl.when(sid < ACC_T)
    def _():
        for k in range(ACC_R // CHUNK):
            pltpu.sync_copy(
                rows_a, acc_sh.at[pl.ds(sid * ACC_R + k * CHUNK, CHUNK)])
        pltpu.sync_copy(rows_a.at[pl.ds(0, ACC_R % CHUNK)],
                        acc_sh.at[pl.ds(sid * ACC_R + ACC_R - ACC_R % CHUNK,
                                        ACC_R % CHUNK)])

    plsc.subcore_barrier()

    ones16 = jnp.full((16,), 1.0, jnp.float32)
    bufs = (rows_a, rows_b, rows_c)
    gsems = (sem_a, sem_b, sem_c)
    ssems = (sem_sa, sem_sb, sem_sc)

    def block(ib, _):
        pltpu.sync_copy(ei_hbm.at[0, wid * NIB + ib], src_v)
        pltpu.sync_copy(ei_hbm.at[1, wid * NIB + ib], dst_v)
        pltpu.async_copy(x_hbm.at[src_v.at[pl.ds(0, CHUNK)]], rows_a, sem_a)

        def step(j, _):
            @pl.when(j + 1 < IBLK)
            def _():
                for b in range(3):
                    @pl.when(lax.rem(j + 1, 3) == b)
                    def _(b=b):
                        @pl.when(j >= 2)
                        def _(b=b):
                            pltpu.make_async_copy(
                                bufs[b],
                                acc_sh.at[dst_v.at[pl.ds(0, CHUNK)]],
                                ssems[b]).wait()
                        pltpu.async_copy(
                            x_hbm.at[src_v.at[pl.ds((j + 1) * CHUNK, CHUNK)]],
                            bufs[b], gsems[b])

            for k in range(CHUNK // 16):
                idx = dst_v[pl.ds(j * CHUNK + k * 16, 16)]
                plsc.addupdate_scatter(hist_v, [idx], ones16)

            for b in range(3):
                @pl.when(lax.rem(j, 3) == b)
                def _(b=b):
                    pltpu.make_async_copy(
                        x_hbm.at[src_v.at[pl.ds(j * CHUNK, CHUNK)]],
                        bufs[b], gsems[b]).wait()
                    pltpu.async_copy(
                        bufs[b],
                        acc_sh.at[dst_v.at[pl.ds(j * CHUNK, CHUNK)]],
                        ssems[b], add=True)

            return 0

        lax.fori_loop(0, IBLK, step, 0)
        for b in (1, 2, 0):
            pltpu.make_async_copy(
                bufs[b], acc_sh.at[dst_v.at[pl.ds(0, CHUNK)]],
                ssems[b]).wait()
        return 0

    lax.fori_loop(0, NIB, block, 0)

    def phase(p, _):
        pub = lax.rem(sid + p, NS)
        pltpu.sync_copy(hist_v.at[pl.ds(pub * SEG, SEG)], hists_sh.at[sid])
        plsc.subcore_barrier()
        slot = lax.rem(sid - p + NS, NS)
        pltpu.sync_copy(hists_sh.at[slot], tmp_v)
        for c in range(SEG // 16):
            sl = pl.ds(c * 16, 16)
            red_v[sl] = red_v[sl] + tmp_v[sl]
        plsc.subcore_barrier()
        return 0

    lax.fori_loop(0, NS, phase, 0)

    pltpu.sync_copy(red_v, deg_out.at[cid, pl.ds(sid * SEG, SEG)])

    @pl.when(sid < ACC_T)
    def _():
        pltpu.sync_copy(acc_sh.at[pl.ds(sid * ACC_R, ACC_R)],
                        acc_out.at[cid, pl.ds(sid * ACC_R, ACC_R)])


def _tc_body(p_ref, d_ref, x_ref, w_ref, b_ref, o_ref):
    p = p_ref[0] + p_ref[1]
    dgr = (d_ref[0] + d_ref[1]).reshape(1, -1)
    dg = jnp.transpose(dgr, (1, 0))
    agg = p / jnp.maximum(dg, 1.0)
    h = jnp.dot(agg, w_ref[...], preferred_element_type=jnp.float32) + b_ref[...]
    o_ref[...] = jnp.maximum(h, 0.0) + x_ref[...]


def kernel(x, edge_index, W, b):
    ei = edge_index.astype(jnp.int32).reshape(2, NW * NIB, IBLK * CHUNK)

    mesh = plsc.VectorSubcoreMesh(core_axis_name="c", subcore_axis_name="s")
    acc_p, deg_p = pl.kernel(
        _sc_body,
        out_type=(
            jax.ShapeDtypeStruct((NC, NACC, D), jnp.float32),
            jax.ShapeDtypeStruct((NC, NPAD_H), jnp.float32),
        ),
        mesh=mesh,
        compiler_params=pltpu.CompilerParams(needs_layout_passes=False),
        scratch_types=[
            pltpu.VMEM((IBLK * CHUNK,), jnp.int32),
            pltpu.VMEM((IBLK * CHUNK,), jnp.int32),
            pltpu.VMEM((CHUNK, D), jnp.float32),
            pltpu.VMEM((CHUNK, D), jnp.float32),
            pltpu.VMEM((CHUNK, D), jnp.float32),
            pltpu.VMEM((NPAD_H,), jnp.float32),
            pltpu.VMEM((SEG,), jnp.float32),
            pltpu.VMEM((SEG,), jnp.float32),
            pltpu.VMEM_SHARED((NACC, D), jnp.float32),
            pltpu.VMEM_SHARED((NS, SEG), jnp.float32),
            pltpu.SemaphoreType.DMA,
            pltpu.SemaphoreType.DMA,
            pltpu.SemaphoreType.DMA,
            pltpu.SemaphoreType.DMA,
            pltpu.SemaphoreType.DMA,
            pltpu.SemaphoreType.DMA,
        ],
    )(x, ei)

    R = 1024
    grid = ((N_NODES + R - 1) // R,)
    h = pl.pallas_call(
        _tc_body,
        grid=grid,
        in_specs=[
            pl.BlockSpec((NC, R, D), lambda i: (0, i, 0)),
            pl.BlockSpec((NC, R), lambda i: (0, i)),
            pl.BlockSpec((R, D), lambda i: (i, 0)),
            pl.BlockSpec((D, D), lambda i: (0, 0)),
            pl.BlockSpec((1, D), lambda i: (0, 0)),
        ],
        out_specs=pl.BlockSpec((R, D), lambda i: (i, 0)),
        out_shape=jax.ShapeDtypeStruct((N_NODES, D), jnp.float32),
    )(acc_p, deg_p, x, W, b.reshape(1, D))
    return h
